# baseline (speedup 1.0000x reference)
import functools

import jax
import jax.numpy as jnp
from jax import lax
from jax.experimental import pallas as pl
from jax.experimental.pallas import tpu as pltpu
from jax.experimental.pallas import tpu_sc as plsc

N_NODES = 10000
D_FEAT = 128
N_EDGES = 320000
N_TAR = 4096

NC = 2
NS = 16
LANES = 16

WORDS = 1280
ROWS_PER_PASS = 1250
N_PASS = 4
BLOCK_WORDS = ROWS_PER_PASS * WORDS
SENT_PAD = 128
SPMEM_WORDS = BLOCK_WORDS + SENT_PAD
ZSTRIPE = SPMEM_WORDS // NS
FSTRIPE = BLOCK_WORDS // NS
E_PER_TEC = N_EDGES // NS
MEGA = 2000
N_MEGA = E_PER_TEC // MEGA
MROW = 80
MEGA_ROWS = MEGA // MROW

G_ROWS = 2 * N_TAR
G_PER_TEC = N_TAR // (NC * NS)
GCHUNK = 64
NGCH = G_PER_TEC // GCHUNK

K_PAD = WORDS * 8

ZCHUNK = 4096
FCHUNK = FSTRIPE // 20


def _build_body(src_hbm, dst_hbm, c_hbm, src_v, dst_v, idx_v, val_v, idx2_v,
                val2_v, zbuf, bounce, spmem, sem):
    cid = lax.axis_index("c")
    sid = lax.axis_index("s")

    def zinit(i, _):
        zbuf[pl.ds(i * LANES, LANES)] = jnp.zeros((LANES,), jnp.int32)
        return 0
    lax.fori_loop(0, ZCHUNK // LANES, zinit, 0)

    for p in range(N_PASS):
        zoff = sid * ZSTRIPE
        nzfull = ZSTRIPE // ZCHUNK
        for i in range(nzfull):
            pltpu.sync_copy(zbuf, spmem.at[pl.ds(zoff + i * ZCHUNK, ZCHUNK)])
        pltpu.sync_copy(zbuf.at[pl.ds(0, ZSTRIPE - nzfull * ZCHUNK)],
                        spmem.at[pl.ds(zoff + nzfull * ZCHUNK,
                                       ZSTRIPE - nzfull * ZCHUNK)])
        plsc.subcore_barrier()

        row_lo = cid * (ROWS_PER_PASS * N_PASS) + p * ROWS_PER_PASS
        row_hi = row_lo + ROWS_PER_PASS
        sent = BLOCK_WORDS + sid * 8

        def do_mega(m, idx_b, val_b, drain):
            ebase = sid * E_PER_TEC + m * MEGA
            pltpu.sync_copy(src_hbm.at[pl.ds(ebase, MEGA)], src_v)
            pltpu.sync_copy(dst_hbm.at[pl.ds(ebase, MEGA)], dst_v)

            def row_body(q, _):
                for v in range(MROW // LANES):
                    r = src_v[pl.ds(q * MROW + v * LANES, LANES)]
                    c = dst_v[pl.ds(q * MROW + v * LANES, LANES)]
                    inr = (r >= row_lo) & (r < row_hi)
                    t = lax.shift_right_logical(
                        lax.shift_right_logical(c, 8) * 13, 6)
                    addr = (r - row_lo) * WORDS + c - t * WORDS
                    addr = jnp.where(inr, addr, sent)
                    val = jnp.where(
                        inr, lax.shift_left(1, lax.shift_left(t, 2)), 0)
                    idx_b[q, pl.ds(v * LANES, LANES)] = addr
                    val_b[q, pl.ds(v * LANES, LANES)] = val
                return 0
            lax.fori_loop(0, MEGA // MROW, row_body, 0)

            if drain is not None:
                didx, dval = drain
                for q in range(MEGA // MROW):
                    pltpu.make_async_copy(
                        dval.at[q], spmem.at[didx.at[q]], sem).wait()
            for q in range(MEGA // MROW):
                pltpu.async_copy(val_b.at[q], spmem.at[idx_b.at[q]], sem,
                                 add=True)

        do_mega(0, idx_v, val_v, None)

        def pair_body(k, _):
            do_mega(2 * k + 1, idx2_v, val2_v, (idx_v, val_v))
            do_mega(2 * k + 2, idx_v, val_v, (idx2_v, val2_v))
            return 0
        lax.fori_loop(0, (N_MEGA - 2) // 2, pair_body, 0)

        do_mega(N_MEGA - 1, idx2_v, val2_v, (idx_v, val_v))
        for q in range(MEGA // MROW):
            pltpu.make_async_copy(
                val2_v.at[q], spmem.at[idx2_v.at[q]], sem).wait()
        plsc.subcore_barrier()

        foff = sid * FSTRIPE
        for f in range(FSTRIPE // FCHUNK):
            pltpu.sync_copy(spmem.at[pl.ds(foff + f * FCHUNK, FCHUNK)],
                            bounce)
            pltpu.sync_copy(
                bounce,
                c_hbm.at[pl.ds(row_lo * WORDS + foff + f * FCHUNK, FCHUNK)])
        plsc.subcore_barrier()


def _gather_body(c_hbm, x_hbm, t0_hbm, t1_hbm, gi_hbm, gj_hbm, xi_hbm, xj_hbm,
                 idx_v, rows_v, xrows_v, sem):
    cid = lax.axis_index("c")
    sid = lax.axis_index("s")
    wid = cid * NS + sid
    base = wid * G_PER_TEC
    for t_hbm, g_hbm, xg_hbm in ((t0_hbm, gi_hbm, xi_hbm),
                                 (t1_hbm, gj_hbm, xj_hbm)):
        for ch in range(NGCH):
            off = base + ch * GCHUNK
            pltpu.sync_copy(t_hbm.at[pl.ds(off, GCHUNK)], idx_v)
            pltpu.async_copy(c_hbm.at[idx_v], rows_v, sem).wait()
            pltpu.sync_copy(rows_v, g_hbm.at[pl.ds(off, GCHUNK)])
            pltpu.async_copy(x_hbm.at[idx_v], xrows_v, sem).wait()
            pltpu.sync_copy(xrows_v, xg_hbm.at[pl.ds(off, GCHUNK)])


def _dense_body(gi_ref, gj_ref, xi_ref, xj_ref, x_ref, w1_ref, w2_ref, b_ref,
                out_ref, sr_ref):
    @pl.when(pl.program_id(0) == 0)
    def _():
        srow = lax.dot_general(w2_ref[...], x_ref[...],
                               (((1,), (1,)), ((), ())),
                               preferred_element_type=jnp.float32)
        sr_ref[...] = jnp.zeros((8, WORDS), jnp.float32)
        for t in range(7):
            sr_ref[t, :] = srow[0, t * WORDS:(t + 1) * WORDS]
        sr_ref[7, :N_NODES - 7 * WORDS] = srow[0, 7 * WORDS:N_NODES]

    gi = gi_ref[...]
    gj = gj_ref[...]
    mask_const = jnp.int32(0x11111111)
    zi = gi | lax.shift_right_logical(gi, 1)
    zi = (zi | lax.shift_right_logical(zi, 2)) & mask_const
    zj = gj | lax.shift_right_logical(gj, 1)
    zj = (zj | lax.shift_right_logical(zj, 2)) & mask_const
    m = zi & zj
    acc = jnp.zeros(gi.shape, jnp.float32)
    for t in range(8):
        bit = lax.shift_right_logical(m, 4 * t) & 1
        acc = acc + bit.astype(jnp.float32) * sr_ref[t, :][None, :]
    cn_term = jnp.sum(acc, axis=1)
    xij = jnp.sum(xi_ref[...] * xj_ref[...] * w1_ref[...], axis=1)
    out_ref[0, 0, :] = cn_term + xij + b_ref[0, 0]


def kernel(x, adj, tar_ei, boolen, W_xslin, b_xslin):
    del boolen
    x = x.astype(jnp.float32)
    adj0 = adj[0].astype(jnp.int32)
    adj1 = adj[1].astype(jnp.int32)
    t0 = tar_ei[0].astype(jnp.int32)
    t1 = tar_ei[1].astype(jnp.int32)
    w1 = W_xslin[0, :D_FEAT].reshape(1, D_FEAT)
    w2 = W_xslin[0, D_FEAT:].reshape(1, D_FEAT)
    b_arr = b_xslin.reshape(1, 1)

    mesh = plsc.VectorSubcoreMesh(core_axis_name="c", subcore_axis_name="s")

    build = pl.kernel(
        _build_body,
        out_type=jax.ShapeDtypeStruct((N_NODES * WORDS,), jnp.int32),
        mesh=mesh,
        scratch_types=[
            pltpu.VMEM((MEGA,), jnp.int32),
            pltpu.VMEM((MEGA,), jnp.int32),
            pltpu.VMEM((MEGA_ROWS, MROW), jnp.int32),
            pltpu.VMEM((MEGA_ROWS, MROW), jnp.int32),
            pltpu.VMEM((MEGA_ROWS, MROW), jnp.int32),
            pltpu.VMEM((MEGA_ROWS, MROW), jnp.int32),
            pltpu.VMEM((ZCHUNK,), jnp.int32),
            pltpu.VMEM((FCHUNK,), jnp.int32),
            pltpu.VMEM_SHARED((SPMEM_WORDS,), jnp.int32),
            pltpu.SemaphoreType.DMA,
        ],
    )
    c_flat = build(adj0, adj1)
    c_2d = c_flat.reshape(N_NODES, WORDS)

    gather = pl.kernel(
        _gather_body,
        out_type=(
            jax.ShapeDtypeStruct((N_TAR, WORDS), jnp.int32),
            jax.ShapeDtypeStruct((N_TAR, WORDS), jnp.int32),
            jax.ShapeDtypeStruct((N_TAR, D_FEAT), jnp.float32),
            jax.ShapeDtypeStruct((N_TAR, D_FEAT), jnp.float32),
        ),
        mesh=mesh,
        scratch_types=[
            pltpu.VMEM((GCHUNK,), jnp.int32),
            pltpu.VMEM((GCHUNK, WORDS), jnp.int32),
            pltpu.VMEM((GCHUNK, D_FEAT), jnp.float32),
            pltpu.SemaphoreType.DMA,
        ],
    )
    gi, gj, xi, xj = gather(c_2d, x, t0, t1)

    EB = 512
    out_blocks = pl.pallas_call(
        _dense_body,
        grid=(N_TAR // EB,),
        in_specs=[
            pl.BlockSpec((EB, WORDS), lambda i: (i, 0)),
            pl.BlockSpec((EB, WORDS), lambda i: (i, 0)),
            pl.BlockSpec((EB, D_FEAT), lambda i: (i, 0)),
            pl.BlockSpec((EB, D_FEAT), lambda i: (i, 0)),
            pl.BlockSpec((N_NODES, D_FEAT), lambda i: (0, 0)),
            pl.BlockSpec((1, D_FEAT), lambda i: (0, 0)),
            pl.BlockSpec((1, D_FEAT), lambda i: (0, 0)),
            pl.BlockSpec((1, 1), lambda i: (0, 0)),
        ],
        out_specs=pl.BlockSpec((1, 1, EB), lambda i: (i, 0, 0)),
        out_shape=jax.ShapeDtypeStruct((N_TAR // EB, 1, EB), jnp.float32),
        scratch_shapes=[pltpu.VMEM((8, WORDS), jnp.float32)],
    )(gi, gj, xi, xj, x, w1, w2, b_arr)
    return out_blocks.reshape(N_TAR, 1)


# speedup vs baseline: 1.0039x; 1.0039x over previous
import functools

import jax
import jax.numpy as jnp
from jax import lax
from jax.experimental import pallas as pl
from jax.experimental.pallas import tpu as pltpu
from jax.experimental.pallas import tpu_sc as plsc

N_NODES = 10000
D_FEAT = 128
N_EDGES = 320000
N_TAR = 4096

NC = 2
NS = 16
LANES = 16

WORDS = 1280
ROWS_PER_PASS = 1250
N_PASS = 4
BLOCK_WORDS = ROWS_PER_PASS * WORDS
SENT_PAD = 128
SPMEM_WORDS = BLOCK_WORDS + SENT_PAD
ZSTRIPE = SPMEM_WORDS // NS
FSTRIPE = BLOCK_WORDS // NS
E_PER_TEC = N_EDGES // NS
MEGA = 2000
N_MEGA = E_PER_TEC // MEGA
MROW = 80
MEGA_ROWS = MEGA // MROW

G_ROWS = 2 * N_TAR
G_PER_TEC = N_TAR // (NC * NS)
GCHUNK = 64
NGCH = G_PER_TEC // GCHUNK

K_PAD = WORDS * 8

ZCHUNK = 4096
FCHUNK = FSTRIPE // 20


def _build_body(src_hbm, dst_hbm, c_hbm, src_v, dst_v, idx_v, val_v, idx2_v,
                val2_v, zbuf, bounce, spmem, sem):
    cid = lax.axis_index("c")
    sid = lax.axis_index("s")

    def zinit(i, _):
        zbuf[pl.ds(i * LANES, LANES)] = jnp.zeros((LANES,), jnp.int32)
        return 0
    lax.fori_loop(0, ZCHUNK // LANES, zinit, 0)

    for p in range(N_PASS):
        zoff = sid * ZSTRIPE
        nzfull = ZSTRIPE // ZCHUNK
        for i in range(nzfull):
            pltpu.sync_copy(zbuf, spmem.at[pl.ds(zoff + i * ZCHUNK, ZCHUNK)])
        pltpu.sync_copy(zbuf.at[pl.ds(0, ZSTRIPE - nzfull * ZCHUNK)],
                        spmem.at[pl.ds(zoff + nzfull * ZCHUNK,
                                       ZSTRIPE - nzfull * ZCHUNK)])
        plsc.subcore_barrier()

        row_lo = cid * (ROWS_PER_PASS * N_PASS) + p * ROWS_PER_PASS
        row_hi = row_lo + ROWS_PER_PASS
        sent = BLOCK_WORDS + sid * 8

        def do_mega(m, idx_b, val_b, drain):
            ebase = sid * E_PER_TEC + m * MEGA
            pltpu.sync_copy(src_hbm.at[pl.ds(ebase, MEGA)], src_v)
            pltpu.sync_copy(dst_hbm.at[pl.ds(ebase, MEGA)], dst_v)

            def row_body(q, _):
                for v in range(MROW // LANES):
                    r = src_v[pl.ds(q * MROW + v * LANES, LANES)]
                    c = dst_v[pl.ds(q * MROW + v * LANES, LANES)]
                    inr = (r >= row_lo) & (r < row_hi)
                    t = lax.shift_right_logical(
                        lax.shift_right_logical(c, 8) * 13, 6)
                    addr = (r - row_lo) * WORDS + c - t * WORDS
                    addr = jnp.where(inr, addr, sent)
                    val = jnp.where(
                        inr, lax.shift_left(1, lax.shift_left(t, 2)), 0)
                    idx_b[q, pl.ds(v * LANES, LANES)] = addr
                    val_b[q, pl.ds(v * LANES, LANES)] = val
                return 0
            lax.fori_loop(0, MEGA // MROW, row_body, 0)

            if drain is not None:
                didx, dval = drain
                for q in range(MEGA // MROW):
                    pltpu.make_async_copy(
                        dval.at[q], spmem.at[didx.at[q]], sem).wait()
            for q in range(MEGA // MROW):
                pltpu.async_copy(val_b.at[q], spmem.at[idx_b.at[q]], sem,
                                 add=True)

        do_mega(0, idx_v, val_v, None)

        def pair_body(k, _):
            do_mega(2 * k + 1, idx2_v, val2_v, (idx_v, val_v))
            do_mega(2 * k + 2, idx_v, val_v, (idx2_v, val2_v))
            return 0
        lax.fori_loop(0, (N_MEGA - 2) // 2, pair_body, 0)

        do_mega(N_MEGA - 1, idx2_v, val2_v, (idx_v, val_v))
        for q in range(MEGA // MROW):
            pltpu.make_async_copy(
                val2_v.at[q], spmem.at[idx2_v.at[q]], sem).wait()
        plsc.subcore_barrier()

        foff = sid * FSTRIPE
        for f in range(FSTRIPE // FCHUNK):
            pltpu.sync_copy(spmem.at[pl.ds(foff + f * FCHUNK, FCHUNK)],
                            bounce)
            pltpu.sync_copy(
                bounce,
                c_hbm.at[pl.ds(row_lo * WORDS + foff + f * FCHUNK, FCHUNK)])
        plsc.subcore_barrier()


def _gather_body(c_hbm, x_hbm, t0_hbm, t1_hbm, gi_hbm, gj_hbm, xi_hbm, xj_hbm,
                 idx_v, rows_v, xrows_v, sem):
    cid = lax.axis_index("c")
    sid = lax.axis_index("s")
    wid = cid * NS + sid
    base = wid * G_PER_TEC
    for t_hbm, g_hbm, xg_hbm in ((t0_hbm, gi_hbm, xi_hbm),
                                 (t1_hbm, gj_hbm, xj_hbm)):
        for ch in range(NGCH):
            off = base + ch * GCHUNK
            pltpu.sync_copy(t_hbm.at[pl.ds(off, GCHUNK)], idx_v)
            pltpu.async_copy(c_hbm.at[idx_v], rows_v, sem).wait()
            pltpu.sync_copy(rows_v, g_hbm.at[pl.ds(off, GCHUNK)])
            pltpu.async_copy(x_hbm.at[idx_v], xrows_v, sem).wait()
            pltpu.sync_copy(xrows_v, xg_hbm.at[pl.ds(off, GCHUNK)])


def _s_body(x_ref, w2_ref, out_ref):
    out_ref[0, 0, :] = jnp.sum(x_ref[...] * w2_ref[...], axis=1)


def _dense_body(gi_ref, gj_ref, xi_ref, xj_ref, sr_ref, w1_ref, b_ref,
                out_ref):
    gi = gi_ref[...]
    gj = gj_ref[...]
    mask_const = jnp.int32(0x11111111)
    zi = gi | lax.shift_right_logical(gi, 1)
    zi = (zi | lax.shift_right_logical(zi, 2)) & mask_const
    zj = gj | lax.shift_right_logical(gj, 1)
    zj = (zj | lax.shift_right_logical(zj, 2)) & mask_const
    m = zi & zj
    acc = jnp.zeros(gi.shape, jnp.float32)
    for t in range(8):
        bit = lax.shift_right_logical(m, 4 * t) & 1
        acc = acc + bit.astype(jnp.float32) * sr_ref[t, :][None, :]
    cn_term = jnp.sum(acc, axis=1)
    xij = jnp.sum(xi_ref[...] * xj_ref[...] * w1_ref[...], axis=1)
    out_ref[0, 0, :] = cn_term + xij + b_ref[0, 0]


def kernel(x, adj, tar_ei, boolen, W_xslin, b_xslin):
    del boolen
    x = x.astype(jnp.float32)
    adj0 = adj[0].astype(jnp.int32)
    adj1 = adj[1].astype(jnp.int32)
    t0 = tar_ei[0].astype(jnp.int32)
    t1 = tar_ei[1].astype(jnp.int32)
    w1 = W_xslin[0, :D_FEAT].reshape(1, D_FEAT)
    w2 = W_xslin[0, D_FEAT:].reshape(1, D_FEAT)
    b_arr = b_xslin.reshape(1, 1)

    mesh = plsc.VectorSubcoreMesh(core_axis_name="c", subcore_axis_name="s")

    build = pl.kernel(
        _build_body,
        out_type=jax.ShapeDtypeStruct((N_NODES * WORDS,), jnp.int32),
        mesh=mesh,
        scratch_types=[
            pltpu.VMEM((MEGA,), jnp.int32),
            pltpu.VMEM((MEGA,), jnp.int32),
            pltpu.VMEM((MEGA_ROWS, MROW), jnp.int32),
            pltpu.VMEM((MEGA_ROWS, MROW), jnp.int32),
            pltpu.VMEM((MEGA_ROWS, MROW), jnp.int32),
            pltpu.VMEM((MEGA_ROWS, MROW), jnp.int32),
            pltpu.VMEM((ZCHUNK,), jnp.int32),
            pltpu.VMEM((FCHUNK,), jnp.int32),
            pltpu.VMEM_SHARED((SPMEM_WORDS,), jnp.int32),
            pltpu.SemaphoreType.DMA,
        ],
    )
    c_flat = build(adj0, adj1)
    c_2d = c_flat.reshape(N_NODES, WORDS)

    gather = pl.kernel(
        _gather_body,
        out_type=(
            jax.ShapeDtypeStruct((N_TAR, WORDS), jnp.int32),
            jax.ShapeDtypeStruct((N_TAR, WORDS), jnp.int32),
            jax.ShapeDtypeStruct((N_TAR, D_FEAT), jnp.float32),
            jax.ShapeDtypeStruct((N_TAR, D_FEAT), jnp.float32),
        ),
        mesh=mesh,
        scratch_types=[
            pltpu.VMEM((GCHUNK,), jnp.int32),
            pltpu.VMEM((GCHUNK, WORDS), jnp.int32),
            pltpu.VMEM((GCHUNK, D_FEAT), jnp.float32),
            pltpu.SemaphoreType.DMA,
        ],
    )
    gi, gj, xi, xj = gather(c_2d, x, t0, t1)

    s_blocks = pl.pallas_call(
        _s_body,
        grid=(N_NODES // 2000,),
        in_specs=[
            pl.BlockSpec((2000, D_FEAT), lambda i: (i, 0)),
            pl.BlockSpec((1, D_FEAT), lambda i: (0, 0)),
        ],
        out_specs=pl.BlockSpec((1, 1, 2000), lambda i: (i, 0, 0)),
        out_shape=jax.ShapeDtypeStruct((N_NODES // 2000, 1, 2000),
                                       jnp.float32),
    )(x, w2)
    s_r = jnp.pad(s_blocks.reshape(N_NODES),
                  (0, K_PAD - N_NODES)).reshape(8, WORDS)

    EB = 512
    out_blocks = pl.pallas_call(
        _dense_body,
        grid=(N_TAR // EB,),
        in_specs=[
            pl.BlockSpec((EB, WORDS), lambda i: (i, 0)),
            pl.BlockSpec((EB, WORDS), lambda i: (i, 0)),
            pl.BlockSpec((EB, D_FEAT), lambda i: (i, 0)),
            pl.BlockSpec((EB, D_FEAT), lambda i: (i, 0)),
            pl.BlockSpec((8, WORDS), lambda i: (0, 0)),
            pl.BlockSpec((1, D_FEAT), lambda i: (0, 0)),
            pl.BlockSpec((1, 1), lambda i: (0, 0)),
        ],
        out_specs=pl.BlockSpec((1, 1, EB), lambda i: (i, 0, 0)),
        out_shape=jax.ShapeDtypeStruct((N_TAR // EB, 1, EB), jnp.float32),
    )(gi, gj, xi, xj, s_r, w1, b_arr)
    return out_blocks.reshape(N_TAR, 1)
